# Initial kernel scaffold; baseline (speedup 1.0000x reference)
import jax
import jax.numpy as jnp
from jax.experimental import pallas as pl


def kernel(x, edge_index, edge_d, WK, bK, WQ, bQ, WV, bV, WdK, bdK, WdV, bdV, WIB, bIB, WS1, bS1, WS2, bS2):
    raise NotImplementedError("write your pallas kernel here")


# speedup vs baseline: 1.0081x; 1.0081x over previous
import functools

import jax
import jax.numpy as jnp
from jax.experimental import pallas as pl
from jax.experimental.pallas import tpu as pltpu

N_NODES = 10000
N_EDGES = 160000
F = 128
CUTOFF = 5.0

_NODE_BLK = 2000


def _node_mm_body(x_ref, wk, bk, wq, bq, wv, bv, ws1, bs1, ws2, bs2,
                  k_ref, q_ref, v_ref, s1_ref, s2_ref):
    x = x_ref[...]
    k_ref[...] = x @ wk[...] + bk[...]
    q_ref[...] = x @ wq[...] + bq[...]
    v = x @ wv[...] + bv[...]
    v_ref[...] = v
    s1_ref[...] = v @ ws1[...] + bs1[...]
    s2_ref[...] = v @ ws2[...] + bs2[...]


def _node_matmuls(x, WK, bK, WQ, bQ, WV, bV, WS1, bS1, WS2, bS2):
    blk = pl.BlockSpec((_NODE_BLK, F), lambda i: (i, 0))
    wspec = pl.BlockSpec((F, F), lambda i: (0, 0))
    bspec = pl.BlockSpec((F,), lambda i: (0,))
    out_shape = [jax.ShapeDtypeStruct((N_NODES, F), jnp.float32)] * 5
    return pl.pallas_call(
        _node_mm_body,
        grid=(N_NODES // _NODE_BLK,),
        in_specs=[blk, wspec, bspec, wspec, bspec, wspec, bspec,
                  wspec, bspec, wspec, bspec],
        out_specs=[blk] * 5,
        out_shape=out_shape,
    )(x, WK, bK, WQ, bQ, WV, bV, WS1, bS1, WS2, bS2)


def kernel(x, edge_index, edge_d, WK, bK, WQ, bQ, WV, bV, WdK, bdK, WdV, bdV,
           WIB, bIB, WS1, bS1, WS2, bS2):
    src = edge_index[0]
    dst = edge_index[1]
    k, q, v, s1, s2 = _node_matmuls(x, WK, bK, WQ, bQ, WV, bV, WS1, bS1, WS2, bS2)

    mu = jnp.linspace(0.0, CUTOFF, F)
    delta = mu[1] - mu[0]
    coeff = -0.5 / (delta ** 2)
    bf_raw = jnp.exp(coeff * (edge_d[:, None] - mu[None, :]) ** 2)
    cut = jnp.where(edge_d < CUTOFF,
                    0.5 * (jnp.cos(jnp.pi * edge_d / CUTOFF) + 1.0), 0.0)
    ev = bf_raw * cut[:, None]
    dV = jax.nn.silu(bf_raw @ WdV + bdV)
    dK = jax.nn.silu(bf_raw @ WdK + bdK)
    weight = jax.nn.silu(jnp.sum(k[src] * q[dst] * dK, axis=-1)) * cut
    value = v[src] * ev * dV * cut[:, None]
    msg = value * weight[:, None]
    h = jax.ops.segment_prod(msg, dst, num_segments=N_NODES)
    y = h @ WIB + bIB
    return (y, s1, s2)


# speedup vs baseline: 2.1314x; 2.1143x over previous
import functools

import jax
import jax.numpy as jnp
from jax import lax
from jax.experimental import pallas as pl
from jax.experimental.pallas import tpu as pltpu
from jax.experimental.pallas import tpu_sc as plsc

N_NODES = 10000
N_EDGES = 160000
F = 128
CUTOFF = 5.0

_NW = 32
_CH = 128
_NCHUNK = N_EDGES // _CH
_GITERS = -(-_NCHUNK // _NW)


def _gather_body(k_hbm, q_hbm, v_hbm, src_hbm, dst_hbm,
                 ks_out, qd_out, vs_out,
                 idx_s, idx_d, kbuf, qbuf, vbuf, sk, sq, sv):
    wid = lax.axis_index("s") * 2 + lax.axis_index("c")

    def body(i, carry):
        c = i * _NW + wid

        @pl.when(c < _NCHUNK)
        def _():
            base = c * _CH
            pltpu.sync_copy(src_hbm.at[pl.ds(base, _CH)], idx_s)
            pltpu.sync_copy(dst_hbm.at[pl.ds(base, _CH)], idx_d)
            ck = pltpu.async_copy(k_hbm.at[idx_s], kbuf, sk)
            cq = pltpu.async_copy(q_hbm.at[idx_d], qbuf, sq)
            cv = pltpu.async_copy(v_hbm.at[idx_s], vbuf, sv)
            ck.wait()
            cq.wait()
            cv.wait()
            pltpu.sync_copy(kbuf, ks_out.at[pl.ds(base, _CH)])
            pltpu.sync_copy(qbuf, qd_out.at[pl.ds(base, _CH)])
            pltpu.sync_copy(vbuf, vs_out.at[pl.ds(base, _CH)])

        return carry

    lax.fori_loop(0, _GITERS, body, 0)


def _sc_gather(k, q, v, src, dst):
    mesh = plsc.VectorSubcoreMesh(core_axis_name="c", subcore_axis_name="s")
    return pl.kernel(
        _gather_body,
        mesh=mesh,
        out_type=[jax.ShapeDtypeStruct((N_EDGES, F), jnp.float32)] * 3,
        scratch_types=(
            [pltpu.VMEM((_CH,), jnp.int32)] * 2
            + [pltpu.VMEM((_CH, F), jnp.float32)] * 3
            + [pltpu.SemaphoreType.DMA] * 3
        ),
    )(k, q, v, src, dst)

_NODE_BLK = 2000


def _node_mm_body(x_ref, wk, bk, wq, bq, wv, bv, ws1, bs1, ws2, bs2,
                  k_ref, q_ref, v_ref, s1_ref, s2_ref):
    x = x_ref[...]
    k_ref[...] = x @ wk[...] + bk[...]
    q_ref[...] = x @ wq[...] + bq[...]
    v = x @ wv[...] + bv[...]
    v_ref[...] = v
    s1_ref[...] = v @ ws1[...] + bs1[...]
    s2_ref[...] = v @ ws2[...] + bs2[...]


def _node_matmuls(x, WK, bK, WQ, bQ, WV, bV, WS1, bS1, WS2, bS2):
    blk = pl.BlockSpec((_NODE_BLK, F), lambda i: (i, 0))
    wspec = pl.BlockSpec((F, F), lambda i: (0, 0))
    bspec = pl.BlockSpec((F,), lambda i: (0,))
    out_shape = [jax.ShapeDtypeStruct((N_NODES, F), jnp.float32)] * 5
    return pl.pallas_call(
        _node_mm_body,
        grid=(N_NODES // _NODE_BLK,),
        in_specs=[blk, wspec, bspec, wspec, bspec, wspec, bspec,
                  wspec, bspec, wspec, bspec],
        out_specs=[blk] * 5,
        out_shape=out_shape,
    )(x, WK, bK, WQ, bQ, WV, bV, WS1, bS1, WS2, bS2)


def kernel(x, edge_index, edge_d, WK, bK, WQ, bQ, WV, bV, WdK, bdK, WdV, bdV,
           WIB, bIB, WS1, bS1, WS2, bS2):
    src = edge_index[0]
    dst = edge_index[1]
    k, q, v, s1, s2 = _node_matmuls(x, WK, bK, WQ, bQ, WV, bV, WS1, bS1, WS2, bS2)

    mu = jnp.linspace(0.0, CUTOFF, F)
    delta = mu[1] - mu[0]
    coeff = -0.5 / (delta ** 2)
    bf_raw = jnp.exp(coeff * (edge_d[:, None] - mu[None, :]) ** 2)
    cut = jnp.where(edge_d < CUTOFF,
                    0.5 * (jnp.cos(jnp.pi * edge_d / CUTOFF) + 1.0), 0.0)
    ev = bf_raw * cut[:, None]
    dV = jax.nn.silu(bf_raw @ WdV + bdV)
    dK = jax.nn.silu(bf_raw @ WdK + bdK)
    ksrc, qdst, vsrc = _sc_gather(k, q, v, src, dst)
    weight = jax.nn.silu(jnp.sum(ksrc * qdst * dK, axis=-1)) * cut
    value = vsrc * ev * dV * cut[:, None]
    msg = value * weight[:, None]
    h = jax.ops.segment_prod(msg, dst, num_segments=N_NODES)
    y = h @ WIB + bIB
    return (y, s1, s2)


# speedup vs baseline: 3.0157x; 1.4149x over previous
import functools

import jax
import jax.numpy as jnp
from jax import lax
from jax.experimental import pallas as pl
from jax.experimental.pallas import tpu as pltpu
from jax.experimental.pallas import tpu_sc as plsc

N_NODES = 10000
N_EDGES = 160000
F = 128
CUTOFF = 5.0

_NW = 32
_CH = 128
_NCHUNK = N_EDGES // _CH
_GITERS = -(-_NCHUNK // _NW)


def _gather_body(k_hbm, q_hbm, v_hbm, src_hbm, dst_hbm,
                 ks_out, qd_out, vs_out,
                 idx_s, idx_d, kbuf, qbuf, vbuf, sk, sq, sv):
    wid = lax.axis_index("s") * 2 + lax.axis_index("c")

    def body(i, carry):
        c = i * _NW + wid

        @pl.when(c < _NCHUNK)
        def _():
            base = c * _CH
            pltpu.sync_copy(src_hbm.at[pl.ds(base, _CH)], idx_s)
            pltpu.sync_copy(dst_hbm.at[pl.ds(base, _CH)], idx_d)
            ck = pltpu.async_copy(k_hbm.at[idx_s], kbuf, sk)
            cq = pltpu.async_copy(q_hbm.at[idx_d], qbuf, sq)
            cv = pltpu.async_copy(v_hbm.at[idx_s], vbuf, sv)
            ck.wait()
            cq.wait()
            cv.wait()
            pltpu.sync_copy(kbuf, ks_out.at[pl.ds(base, _CH)])
            pltpu.sync_copy(qbuf, qd_out.at[pl.ds(base, _CH)])
            pltpu.sync_copy(vbuf, vs_out.at[pl.ds(base, _CH)])

        return carry

    lax.fori_loop(0, _GITERS, body, 0)


def _sc_gather(k, q, v, src, dst):
    mesh = plsc.VectorSubcoreMesh(core_axis_name="c", subcore_axis_name="s")
    return pl.kernel(
        _gather_body,
        mesh=mesh,
        out_type=[jax.ShapeDtypeStruct((N_EDGES, F), jnp.float32)] * 3,
        scratch_types=(
            [pltpu.VMEM((_CH,), jnp.int32)] * 2
            + [pltpu.VMEM((_CH, F), jnp.float32)] * 3
            + [pltpu.SemaphoreType.DMA] * 3
        ),
    )(k, q, v, src, dst)

_NODE_BLK = 2000


def _node_mm_body(x_ref, wk, bk, wq, bq, wv, bv, ws1, bs1, ws2, bs2,
                  k_ref, q_ref, v_ref, s1_ref, s2_ref):
    x = x_ref[...]
    k_ref[...] = x @ wk[...] + bk[...]
    q_ref[...] = x @ wq[...] + bq[...]
    v = x @ wv[...] + bv[...]
    v_ref[...] = v
    s1_ref[...] = v @ ws1[...] + bs1[...]
    s2_ref[...] = v @ ws2[...] + bs2[...]


def _node_matmuls(x, WK, bK, WQ, bQ, WV, bV, WS1, bS1, WS2, bS2):
    blk = pl.BlockSpec((_NODE_BLK, F), lambda i: (i, 0))
    wspec = pl.BlockSpec((F, F), lambda i: (0, 0))
    bspec = pl.BlockSpec((F,), lambda i: (0,))
    out_shape = [jax.ShapeDtypeStruct((N_NODES, F), jnp.float32)] * 5
    return pl.pallas_call(
        _node_mm_body,
        grid=(N_NODES // _NODE_BLK,),
        in_specs=[blk, wspec, bspec, wspec, bspec, wspec, bspec,
                  wspec, bspec, wspec, bspec],
        out_specs=[blk] * 5,
        out_shape=out_shape,
    )(x, WK, bK, WQ, bQ, WV, bV, WS1, bS1, WS2, bS2)


_EDGE_BLK = 3200


def _edge_body(d_ref, ks_ref, qd_ref, vs_ref, mu_ref,
               wdk, bdk, wdv, bdv, logabs_ref, neg_ref):
    d = d_ref[...][:, 0]
    mu = mu_ref[...]
    delta = mu[1] - mu[0]
    coeff = -0.5 / (delta ** 2)
    bf = jnp.exp(coeff * (d[:, None] - mu[None, :]) ** 2)
    cut = jnp.where(d < CUTOFF,
                    0.5 * (jnp.cos(jnp.pi * d / CUTOFF) + 1.0), 0.0)
    dVb = jax.nn.silu(bf @ wdv[...] + bdv[...])
    dKb = jax.nn.silu(bf @ wdk[...] + bdk[...])
    weight = jax.nn.silu(
        jnp.sum(ks_ref[...] * qd_ref[...] * dKb, axis=-1)) * cut
    ev = bf * cut[:, None]
    value = vs_ref[...] * ev * dVb * cut[:, None]
    msg = value * weight[:, None]
    logabs_ref[...] = jnp.log(jnp.abs(msg))
    neg_ref[...] = (msg < 0).astype(jnp.float32)


def _edge_stage(edge_d, ksrc, qdst, vsrc, mu, WdK, bdK, WdV, bdV):
    eblk = pl.BlockSpec((_EDGE_BLK, F), lambda i: (i, 0))
    dblk = pl.BlockSpec((_EDGE_BLK, 1), lambda i: (i, 0))
    wspec = pl.BlockSpec((F, F), lambda i: (0, 0))
    bspec = pl.BlockSpec((F,), lambda i: (0,))
    return pl.pallas_call(
        _edge_body,
        grid=(N_EDGES // _EDGE_BLK,),
        in_specs=[dblk, eblk, eblk, eblk, bspec, wspec, bspec, wspec, bspec],
        out_specs=[eblk, eblk],
        out_shape=[jax.ShapeDtypeStruct((N_EDGES, F), jnp.float32),
                   jax.ShapeDtypeStruct((N_EDGES, F), jnp.float32)],
    )(edge_d[:, None], ksrc, qdst, vsrc, mu, WdK, bdK, WdV, bdV)


_NPAD = 10240
_NSUB = _NPAD // 16
_SITERS = -(-_NCHUNK // 16)


def _scatter_body(logabs_hbm, neg_hbm, dst_hbm, zf_hbm,
                  acc_out,
                  idx, lbuf, acc):
    core = lax.axis_index("c")
    sub = lax.axis_index("s")
    pltpu.sync_copy(zf_hbm.at[pl.ds(sub * _NSUB, _NSUB)],
                    acc.at[pl.ds(sub * _NSUB, _NSUB)])
    plsc.subcore_barrier()

    def body(i, carry):
        c = i * 16 + sub

        @pl.when(c < _NCHUNK)
        def _():
            base = c * _CH
            pltpu.sync_copy(dst_hbm.at[pl.ds(base, _CH)], idx)

            @pl.when(core == 0)
            def _():
                pltpu.sync_copy(logabs_hbm.at[pl.ds(base, _CH)], lbuf)

            @pl.when(core == 1)
            def _():
                pltpu.sync_copy(neg_hbm.at[pl.ds(base, _CH)], lbuf)

            pltpu.sync_copy(lbuf, acc.at[idx], add=True)

        return carry

    lax.fori_loop(0, _SITERS, body, 0)
    plsc.subcore_barrier()
    pltpu.sync_copy(acc.at[pl.ds(sub * _NSUB, _NSUB)],
                    acc_out.at[core, pl.ds(sub * _NSUB, _NSUB)])


def _sc_scatter(logabs, neg, dst, zf):
    mesh = plsc.VectorSubcoreMesh(core_axis_name="c", subcore_axis_name="s")
    return pl.kernel(
        _scatter_body,
        mesh=mesh,
        out_type=jax.ShapeDtypeStruct((2, _NPAD, F), jnp.float32),
        scratch_types=[
            pltpu.VMEM((_CH,), jnp.int32),
            pltpu.VMEM((_CH, F), jnp.float32),
            pltpu.VMEM_SHARED((_NPAD, F), jnp.float32),
        ],
    )(logabs, neg, dst, zf)


def _final_body(ls_ref, cnt_ref, wib, bib, y_ref):
    ls = ls_ref[...]
    cnt = cnt_ref[...]
    parity = cnt - 2.0 * jnp.floor(cnt * 0.5)
    sign = 1.0 - 2.0 * parity
    h = sign * jnp.exp(ls)
    y_ref[...] = h @ wib[...] + bib[...]


def _final_stage(logsum, cnt, WIB, bIB):
    blk = pl.BlockSpec((_NODE_BLK, F), lambda i: (i, 0))
    wspec = pl.BlockSpec((F, F), lambda i: (0, 0))
    bspec = pl.BlockSpec((F,), lambda i: (0,))
    return pl.pallas_call(
        _final_body,
        grid=(N_NODES // _NODE_BLK,),
        in_specs=[blk, blk, wspec, bspec],
        out_specs=blk,
        out_shape=jax.ShapeDtypeStruct((N_NODES, F), jnp.float32),
    )(logsum, cnt, WIB, bIB)


def kernel(x, edge_index, edge_d, WK, bK, WQ, bQ, WV, bV, WdK, bdK, WdV, bdV,
           WIB, bIB, WS1, bS1, WS2, bS2):
    src = edge_index[0]
    dst = edge_index[1]
    k, q, v, s1, s2 = _node_matmuls(x, WK, bK, WQ, bQ, WV, bV, WS1, bS1, WS2, bS2)
    ksrc, qdst, vsrc = _sc_gather(k, q, v, src, dst)

    mu = jnp.linspace(0.0, CUTOFF, F)
    logabs, neg = _edge_stage(edge_d, ksrc, qdst, vsrc, mu, WdK, bdK, WdV, bdV)

    zf = jnp.zeros((_NPAD, F), jnp.float32)
    accs = _sc_scatter(logabs, neg, dst, zf)
    y = _final_stage(accs[0, :N_NODES], accs[1, :N_NODES], WIB, bIB)
    return (y, s1, s2)


# speedup vs baseline: 3.2061x; 1.0631x over previous
import functools

import jax
import jax.numpy as jnp
from jax import lax
from jax.experimental import pallas as pl
from jax.experimental.pallas import tpu as pltpu
from jax.experimental.pallas import tpu_sc as plsc

N_NODES = 10000
N_EDGES = 160000
F = 128
CUTOFF = 5.0

_NW = 32
_CH = 128
_NCHUNK = N_EDGES // _CH
_GITERS = -(-_NCHUNK // _NW)


def _gather_body(x_hbm, src_hbm, dst_hbm,
                 xs_out, xd_out,
                 idx_s, idx_d, sbuf, dbuf, ss, sd):
    wid = lax.axis_index("s") * 2 + lax.axis_index("c")

    def body(i, carry):
        c = i * _NW + wid

        @pl.when(c < _NCHUNK)
        def _():
            base = c * _CH
            pltpu.sync_copy(src_hbm.at[pl.ds(base, _CH)], idx_s)
            pltpu.sync_copy(dst_hbm.at[pl.ds(base, _CH)], idx_d)
            cs = pltpu.async_copy(x_hbm.at[idx_s], sbuf, ss)
            cd = pltpu.async_copy(x_hbm.at[idx_d], dbuf, sd)
            cs.wait()
            cd.wait()
            pltpu.sync_copy(sbuf, xs_out.at[pl.ds(base, _CH)])
            pltpu.sync_copy(dbuf, xd_out.at[pl.ds(base, _CH)])

        return carry

    lax.fori_loop(0, _GITERS, body, 0)


def _sc_gather(x, src, dst):
    mesh = plsc.VectorSubcoreMesh(core_axis_name="c", subcore_axis_name="s")
    return pl.kernel(
        _gather_body,
        mesh=mesh,
        out_type=[jax.ShapeDtypeStruct((N_EDGES, F), jnp.float32)] * 2,
        scratch_types=(
            [pltpu.VMEM((_CH,), jnp.int32)] * 2
            + [pltpu.VMEM((_CH, F), jnp.float32)] * 2
            + [pltpu.SemaphoreType.DMA] * 2
        ),
    )(x, src, dst)

_NODE_BLK = 2000


def _node_mm_body(x_ref, wv, bv, ws1, bs1, ws2, bs2, s1_ref, s2_ref):
    v = x_ref[...] @ wv[...] + bv[...]
    s1_ref[...] = v @ ws1[...] + bs1[...]
    s2_ref[...] = v @ ws2[...] + bs2[...]


def _node_matmuls(x, WV, bV, WS1, bS1, WS2, bS2):
    blk = pl.BlockSpec((_NODE_BLK, F), lambda i: (i, 0))
    wspec = pl.BlockSpec((F, F), lambda i: (0, 0))
    bspec = pl.BlockSpec((F,), lambda i: (0,))
    out_shape = [jax.ShapeDtypeStruct((N_NODES, F), jnp.float32)] * 2
    return pl.pallas_call(
        _node_mm_body,
        grid=(N_NODES // _NODE_BLK,),
        in_specs=[blk, wspec, bspec, wspec, bspec, wspec, bspec],
        out_specs=[blk] * 2,
        out_shape=out_shape,
    )(x, WV, bV, WS1, bS1, WS2, bS2)


_EDGE_BLK = 3200


def _edge_body(d_ref, xs_ref, xd_ref, mu_ref,
               wk, bk, wq, bq, wv, bv,
               wdk, bdk, wdv, bdv, logabs_ref, neg_ref):
    d = d_ref[...][:, 0]
    mu = mu_ref[...]
    delta = mu[1] - mu[0]
    coeff = -0.5 / (delta ** 2)
    bf = jnp.exp(coeff * (d[:, None] - mu[None, :]) ** 2)
    cut = jnp.where(d < CUTOFF,
                    0.5 * (jnp.cos(jnp.pi * d / CUTOFF) + 1.0), 0.0)
    xs = xs_ref[...]
    xd = xd_ref[...]
    ksrc = xs @ wk[...] + bk[...]
    qdst = xd @ wq[...] + bq[...]
    vsrc = xs @ wv[...] + bv[...]
    dVb = jax.nn.silu(bf @ wdv[...] + bdv[...])
    dKb = jax.nn.silu(bf @ wdk[...] + bdk[...])
    weight = jax.nn.silu(jnp.sum(ksrc * qdst * dKb, axis=-1)) * cut
    ev = bf * cut[:, None]
    value = vsrc * ev * dVb * cut[:, None]
    msg = value * weight[:, None]
    logabs_ref[...] = jnp.log(jnp.abs(msg))
    neg_ref[...] = (msg < 0).astype(jnp.float32)


def _edge_stage(edge_d, xsrc, xdst, mu, WK, bK, WQ, bQ, WV, bV,
                WdK, bdK, WdV, bdV):
    eblk = pl.BlockSpec((_EDGE_BLK, F), lambda i: (i, 0))
    dblk = pl.BlockSpec((_EDGE_BLK, 1), lambda i: (i, 0))
    wspec = pl.BlockSpec((F, F), lambda i: (0, 0))
    bspec = pl.BlockSpec((F,), lambda i: (0,))
    return pl.pallas_call(
        _edge_body,
        grid=(N_EDGES // _EDGE_BLK,),
        in_specs=[dblk, eblk, eblk, bspec, wspec, bspec, wspec, bspec,
                  wspec, bspec, wspec, bspec, wspec, bspec],
        out_specs=[eblk, eblk],
        out_shape=[jax.ShapeDtypeStruct((N_EDGES, F), jnp.float32),
                   jax.ShapeDtypeStruct((N_EDGES, F), jnp.float32)],
    )(edge_d[:, None], xsrc, xdst, mu, WK, bK, WQ, bQ, WV, bV,
      WdK, bdK, WdV, bdV)


_NPAD = 10240
_NSUB = _NPAD // 16
_SITERS = -(-_NCHUNK // 16)


def _scatter_body(logabs_hbm, neg_hbm, dst_hbm, zf_hbm,
                  acc_out,
                  idx, lbuf, acc):
    core = lax.axis_index("c")
    sub = lax.axis_index("s")
    pltpu.sync_copy(zf_hbm.at[pl.ds(sub * _NSUB, _NSUB)],
                    acc.at[pl.ds(sub * _NSUB, _NSUB)])
    plsc.subcore_barrier()

    def body(i, carry):
        c = i * 16 + sub

        @pl.when(c < _NCHUNK)
        def _():
            base = c * _CH
            pltpu.sync_copy(dst_hbm.at[pl.ds(base, _CH)], idx)

            @pl.when(core == 0)
            def _():
                pltpu.sync_copy(logabs_hbm.at[pl.ds(base, _CH)], lbuf)

            @pl.when(core == 1)
            def _():
                pltpu.sync_copy(neg_hbm.at[pl.ds(base, _CH)], lbuf)

            pltpu.sync_copy(lbuf, acc.at[idx], add=True)

        return carry

    lax.fori_loop(0, _SITERS, body, 0)
    plsc.subcore_barrier()
    pltpu.sync_copy(acc.at[pl.ds(sub * _NSUB, _NSUB)],
                    acc_out.at[core, pl.ds(sub * _NSUB, _NSUB)])


def _sc_scatter(logabs, neg, dst, zf):
    mesh = plsc.VectorSubcoreMesh(core_axis_name="c", subcore_axis_name="s")
    return pl.kernel(
        _scatter_body,
        mesh=mesh,
        out_type=jax.ShapeDtypeStruct((2, _NPAD, F), jnp.float32),
        scratch_types=[
            pltpu.VMEM((_CH,), jnp.int32),
            pltpu.VMEM((_CH, F), jnp.float32),
            pltpu.VMEM_SHARED((_NPAD, F), jnp.float32),
        ],
    )(logabs, neg, dst, zf)


def _final_body(ls_ref, cnt_ref, wib, bib, y_ref):
    ls = ls_ref[...]
    cnt = cnt_ref[...]
    parity = cnt - 2.0 * jnp.floor(cnt * 0.5)
    sign = 1.0 - 2.0 * parity
    h = sign * jnp.exp(ls)
    y_ref[...] = h @ wib[...] + bib[...]


def _final_stage(logsum, cnt, WIB, bIB):
    blk = pl.BlockSpec((_NODE_BLK, F), lambda i: (i, 0))
    wspec = pl.BlockSpec((F, F), lambda i: (0, 0))
    bspec = pl.BlockSpec((F,), lambda i: (0,))
    return pl.pallas_call(
        _final_body,
        grid=(N_NODES // _NODE_BLK,),
        in_specs=[blk, blk, wspec, bspec],
        out_specs=blk,
        out_shape=jax.ShapeDtypeStruct((N_NODES, F), jnp.float32),
    )(logsum, cnt, WIB, bIB)


def kernel(x, edge_index, edge_d, WK, bK, WQ, bQ, WV, bV, WdK, bdK, WdV, bdV,
           WIB, bIB, WS1, bS1, WS2, bS2):
    src = edge_index[0]
    dst = edge_index[1]
    s1, s2 = _node_matmuls(x, WV, bV, WS1, bS1, WS2, bS2)
    xsrc, xdst = _sc_gather(x, src, dst)

    mu = jnp.linspace(0.0, CUTOFF, F)
    logabs, neg = _edge_stage(edge_d, xsrc, xdst, mu, WK, bK, WQ, bQ, WV, bV,
                              WdK, bdK, WdV, bdV)

    zf = jnp.zeros((_NPAD, F), jnp.float32)
    accs = _sc_scatter(logabs, neg, dst, zf)
    y = _final_stage(accs[0, :N_NODES], accs[1, :N_NODES], WIB, bIB)
    return (y, s1, s2)


# speedup vs baseline: 3.8488x; 1.2005x over previous
import functools

import jax
import jax.numpy as jnp
from jax import lax
from jax.experimental import pallas as pl
from jax.experimental.pallas import tpu as pltpu
from jax.experimental.pallas import tpu_sc as plsc

N_NODES = 10000
N_EDGES = 160000
F = 128
CUTOFF = 5.0

_NW = 32
_CH = 128
_NCHUNK = N_EDGES // _CH
_GITERS = -(-_NCHUNK // _NW)


def _make_gather_body(nchunk):
    giters = -(-nchunk // _NW)

    def _gather_body(x_hbm, src_hbm, dst_hbm,
                     xs_out, xd_out,
                     idx_s, idx_d, sbuf, dbuf, ss, sd):
        wid = lax.axis_index("s") * 2 + lax.axis_index("c")

        def body(i, carry):
            c = i * _NW + wid

            @pl.when(c < nchunk)
            def _():
                base = c * _CH
                pltpu.sync_copy(src_hbm.at[pl.ds(base, _CH)], idx_s)
                pltpu.sync_copy(dst_hbm.at[pl.ds(base, _CH)], idx_d)
                cs = pltpu.async_copy(x_hbm.at[idx_s], sbuf, ss)
                cd = pltpu.async_copy(x_hbm.at[idx_d], dbuf, sd)
                cs.wait()
                cd.wait()
                pltpu.sync_copy(sbuf, xs_out.at[pl.ds(base, _CH)])
                pltpu.sync_copy(dbuf, xd_out.at[pl.ds(base, _CH)])

            return carry

        lax.fori_loop(0, giters, body, 0)

    return _gather_body


def _sc_gather(x, src, dst):
    n_edges = src.shape[0]
    mesh = plsc.VectorSubcoreMesh(core_axis_name="c", subcore_axis_name="s")
    return pl.kernel(
        _make_gather_body(n_edges // _CH),
        mesh=mesh,
        out_type=[jax.ShapeDtypeStruct((n_edges, F), jnp.float32)] * 2,
        scratch_types=(
            [pltpu.VMEM((_CH,), jnp.int32)] * 2
            + [pltpu.VMEM((_CH, F), jnp.float32)] * 2
            + [pltpu.SemaphoreType.DMA] * 2
        ),
    )(x, src, dst)

_NODE_BLK = 2000


def _node_mm_body(x_ref, wv, bv, ws1, bs1, ws2, bs2, s1_ref, s2_ref):
    v = x_ref[...] @ wv[...] + bv[...]
    s1_ref[...] = v @ ws1[...] + bs1[...]
    s2_ref[...] = v @ ws2[...] + bs2[...]


def _node_matmuls(x, WV, bV, WS1, bS1, WS2, bS2):
    blk = pl.BlockSpec((_NODE_BLK, F), lambda i: (i, 0))
    wspec = pl.BlockSpec((F, F), lambda i: (0, 0))
    bspec = pl.BlockSpec((F,), lambda i: (0,))
    out_shape = [jax.ShapeDtypeStruct((N_NODES, F), jnp.float32)] * 2
    return pl.pallas_call(
        _node_mm_body,
        grid=(N_NODES // _NODE_BLK,),
        in_specs=[blk, wspec, bspec, wspec, bspec, wspec, bspec],
        out_specs=[blk] * 2,
        out_shape=out_shape,
    )(x, WV, bV, WS1, bS1, WS2, bS2)


_EDGE_BLK = 3200


def _edge_body(d_ref, xs_ref, xd_ref, mu_ref,
               wk, bk, wq, bq, wv, bv,
               wdk, bdk, wdv, bdv, logabs_ref, neg_ref):
    d = d_ref[...][:, 0]
    mu = mu_ref[...]
    delta = mu[1] - mu[0]
    coeff = -0.5 / (delta ** 2)
    bf = jnp.exp(coeff * (d[:, None] - mu[None, :]) ** 2)
    cut = jnp.where(d < CUTOFF,
                    0.5 * (jnp.cos(jnp.pi * d / CUTOFF) + 1.0), 0.0)
    xs = xs_ref[...]
    xd = xd_ref[...]
    ksrc = xs @ wk[...] + bk[...]
    qdst = xd @ wq[...] + bq[...]
    vsrc = xs @ wv[...] + bv[...]
    dVb = jax.nn.silu(bf @ wdv[...] + bdv[...])
    dKb = jax.nn.silu(bf @ wdk[...] + bdk[...])
    weight = jax.nn.silu(jnp.sum(ksrc * qdst * dKb, axis=-1)) * cut
    ev = bf * cut[:, None]
    value = vsrc * ev * dVb * cut[:, None]
    msg = value * weight[:, None]
    logabs_ref[...] = jnp.log(jnp.abs(msg))
    neg_ref[...] = (msg < 0).astype(jnp.float32)


def _edge_stage(edge_d, xsrc, xdst, mu, WK, bK, WQ, bQ, WV, bV,
                WdK, bdK, WdV, bdV):
    eblk = pl.BlockSpec((_EDGE_BLK, F), lambda i: (i, 0))
    dblk = pl.BlockSpec((_EDGE_BLK, 1), lambda i: (i, 0))
    wspec = pl.BlockSpec((F, F), lambda i: (0, 0))
    bspec = pl.BlockSpec((F,), lambda i: (0,))
    n_edges = xsrc.shape[0]
    return pl.pallas_call(
        _edge_body,
        grid=(n_edges // _EDGE_BLK,),
        in_specs=[dblk, eblk, eblk, bspec, wspec, bspec, wspec, bspec,
                  wspec, bspec, wspec, bspec, wspec, bspec],
        out_specs=[eblk, eblk],
        out_shape=[jax.ShapeDtypeStruct((n_edges, F), jnp.float32),
                   jax.ShapeDtypeStruct((n_edges, F), jnp.float32)],
    )(edge_d[:, None], xsrc, xdst, mu, WK, bK, WQ, bQ, WV, bV,
      WdK, bdK, WdV, bdV)


_NPAD = 10240
_NSUB = _NPAD // 16
_SITERS = -(-_NCHUNK // 16)


def _make_scatter_body(nchunk):
    siters = -(-nchunk // 16)

    def _scatter_body(logabs_hbm, neg_hbm, dst_hbm, zf_hbm,
                      acc_out,
                      idx, lbuf, acc):
        core = lax.axis_index("c")
        sub = lax.axis_index("s")
        pltpu.sync_copy(zf_hbm.at[pl.ds(sub * _NSUB, _NSUB)],
                        acc.at[pl.ds(sub * _NSUB, _NSUB)])
        plsc.subcore_barrier()

        def body(i, carry):
            c = i * 16 + sub

            @pl.when(c < nchunk)
            def _():
                base = c * _CH
                pltpu.sync_copy(dst_hbm.at[pl.ds(base, _CH)], idx)

                @pl.when(core == 0)
                def _():
                    pltpu.sync_copy(logabs_hbm.at[pl.ds(base, _CH)], lbuf)

                @pl.when(core == 1)
                def _():
                    pltpu.sync_copy(neg_hbm.at[pl.ds(base, _CH)], lbuf)

                pltpu.sync_copy(lbuf, acc.at[idx], add=True)

            return carry

        lax.fori_loop(0, siters, body, 0)
        plsc.subcore_barrier()
        pltpu.sync_copy(acc.at[pl.ds(sub * _NSUB, _NSUB)],
                        acc_out.at[core, pl.ds(sub * _NSUB, _NSUB)])

    return _scatter_body


def _sc_scatter(logabs, neg, dst, zf):
    n_edges = dst.shape[0]
    mesh = plsc.VectorSubcoreMesh(core_axis_name="c", subcore_axis_name="s")
    return pl.kernel(
        _make_scatter_body(n_edges // _CH),
        mesh=mesh,
        out_type=jax.ShapeDtypeStruct((2, _NPAD, F), jnp.float32),
        scratch_types=[
            pltpu.VMEM((_CH,), jnp.int32),
            pltpu.VMEM((_CH, F), jnp.float32),
            pltpu.VMEM_SHARED((_NPAD, F), jnp.float32),
        ],
    )(logabs, neg, dst, zf)


def _final_body(ls0_ref, ls1_ref, cnt0_ref, cnt1_ref, wib, bib, y_ref):
    ls = ls0_ref[...] + ls1_ref[...]
    cnt = cnt0_ref[...] + cnt1_ref[...]
    parity = cnt - 2.0 * jnp.floor(cnt * 0.5)
    sign = 1.0 - 2.0 * parity
    h = sign * jnp.exp(ls)
    y_ref[...] = h @ wib[...] + bib[...]


def _final_stage(acc0, acc1, WIB, bIB):
    blk = pl.BlockSpec((_NODE_BLK, F), lambda i: (i, 0))
    wspec = pl.BlockSpec((F, F), lambda i: (0, 0))
    bspec = pl.BlockSpec((F,), lambda i: (0,))
    return pl.pallas_call(
        _final_body,
        grid=(N_NODES // _NODE_BLK,),
        in_specs=[blk, blk, blk, blk, wspec, bspec],
        out_specs=blk,
        out_shape=jax.ShapeDtypeStruct((N_NODES, F), jnp.float32),
    )(acc0[0, :N_NODES], acc1[0, :N_NODES],
      acc0[1, :N_NODES], acc1[1, :N_NODES], WIB, bIB)


def kernel(x, edge_index, edge_d, WK, bK, WQ, bQ, WV, bV, WdK, bdK, WdV, bdV,
           WIB, bIB, WS1, bS1, WS2, bS2):
    src = edge_index[0]
    dst = edge_index[1]
    s1, s2 = _node_matmuls(x, WV, bV, WS1, bS1, WS2, bS2)

    mu = jnp.linspace(0.0, CUTOFF, F)
    zf = jnp.zeros((_NPAD, F), jnp.float32)
    half = N_EDGES // 2
    acc_halves = []
    for lo in (0, half):
        s_h = lax.dynamic_slice_in_dim(src, lo, half)
        d_h = lax.dynamic_slice_in_dim(dst, lo, half)
        e_h = lax.dynamic_slice_in_dim(edge_d, lo, half)
        xsrc, xdst = _sc_gather(x, s_h, d_h)
        logabs, neg = _edge_stage(e_h, xsrc, xdst, mu, WK, bK, WQ, bQ,
                                  WV, bV, WdK, bdK, WdV, bdV)
        acc_halves.append(_sc_scatter(logabs, neg, d_h, zf))
    y = _final_stage(acc_halves[0], acc_halves[1], WIB, bIB)
    return (y, s1, s2)


# speedup vs baseline: 4.1121x; 1.0684x over previous
import functools

import jax
import jax.numpy as jnp
from jax import lax
from jax.experimental import pallas as pl
from jax.experimental.pallas import tpu as pltpu
from jax.experimental.pallas import tpu_sc as plsc

N_NODES = 10000
N_EDGES = 160000
F = 128
CUTOFF = 5.0

_NW = 32
_CH = 128
_NCHUNK = N_EDGES // _CH
_GITERS = -(-_NCHUNK // _NW)


def _make_gather_body(nchunk):
    giters = -(-nchunk // _NW)

    def _gather_body(x_hbm, src_hbm, dst_hbm,
                     xs_out, xd_out,
                     idx_s, idx_d, sbuf, dbuf, ss, sd):
        wid = lax.axis_index("s") * 2 + lax.axis_index("c")

        def body(i, carry):
            c = i * _NW + wid

            @pl.when(c < nchunk)
            def _():
                base = c * _CH
                pltpu.sync_copy(src_hbm.at[pl.ds(base, _CH)], idx_s)
                pltpu.sync_copy(dst_hbm.at[pl.ds(base, _CH)], idx_d)
                cs = pltpu.async_copy(x_hbm.at[idx_s], sbuf, ss)
                cd = pltpu.async_copy(x_hbm.at[idx_d], dbuf, sd)
                cs.wait()
                cd.wait()
                pltpu.sync_copy(sbuf, xs_out.at[pl.ds(base, _CH)])
                pltpu.sync_copy(dbuf, xd_out.at[pl.ds(base, _CH)])

            return carry

        lax.fori_loop(0, giters, body, 0)

    return _gather_body


def _sc_gather(x, src, dst):
    n_edges = src.shape[0]
    mesh = plsc.VectorSubcoreMesh(core_axis_name="c", subcore_axis_name="s")
    return pl.kernel(
        _make_gather_body(n_edges // _CH),
        mesh=mesh,
        out_type=[jax.ShapeDtypeStruct((n_edges, F), jnp.float32)] * 2,
        scratch_types=(
            [pltpu.VMEM((_CH,), jnp.int32)] * 2
            + [pltpu.VMEM((_CH, F), jnp.float32)] * 2
            + [pltpu.SemaphoreType.DMA] * 2
        ),
    )(x, src, dst)

_NODE_BLK = 2000


def _node_mm_body(x_ref, wv, bv, ws1, bs1, ws2, bs2, s1_ref, s2_ref):
    v = x_ref[...] @ wv[...] + bv[...]
    s1_ref[...] = v @ ws1[...] + bs1[...]
    s2_ref[...] = v @ ws2[...] + bs2[...]


def _node_matmuls(x, WV, bV, WS1, bS1, WS2, bS2):
    blk = pl.BlockSpec((_NODE_BLK, F), lambda i: (i, 0))
    wspec = pl.BlockSpec((F, F), lambda i: (0, 0))
    bspec = pl.BlockSpec((F,), lambda i: (0,))
    out_shape = [jax.ShapeDtypeStruct((N_NODES, F), jnp.float32)] * 2
    return pl.pallas_call(
        _node_mm_body,
        grid=(N_NODES // _NODE_BLK,),
        in_specs=[blk, wspec, bspec, wspec, bspec, wspec, bspec],
        out_specs=[blk] * 2,
        out_shape=out_shape,
    )(x, WV, bV, WS1, bS1, WS2, bS2)


_EDGE_BLK = 3200
_NPARTS = 5


def _edge_body(d_ref, xs_ref, xd_ref, mu_ref,
               wk, bk, wq, bq, wv, bv,
               wdk, bdk, wdv, bdv, logabs_ref, neg_ref):
    d = d_ref[...][:, 0]
    mu = mu_ref[...]
    delta = mu[1] - mu[0]
    coeff = -0.5 / (delta ** 2)
    bf = jnp.exp(coeff * (d[:, None] - mu[None, :]) ** 2)
    cut = jnp.where(d < CUTOFF,
                    0.5 * (jnp.cos(jnp.pi * d / CUTOFF) + 1.0), 0.0)
    xs = xs_ref[...]
    xd = xd_ref[...]
    ksrc = xs @ wk[...] + bk[...]
    qdst = xd @ wq[...] + bq[...]
    vsrc = xs @ wv[...] + bv[...]
    dVb = jax.nn.silu(bf @ wdv[...] + bdv[...])
    dKb = jax.nn.silu(bf @ wdk[...] + bdk[...])
    weight = jax.nn.silu(jnp.sum(ksrc * qdst * dKb, axis=-1)) * cut
    ev = bf * cut[:, None]
    value = vsrc * ev * dVb * cut[:, None]
    msg = value * weight[:, None]
    logabs_ref[...] = jnp.log(jnp.abs(msg))
    neg_ref[...] = (msg < 0).astype(jnp.float32)


def _edge_stage(edge_d, xsrc, xdst, mu, WK, bK, WQ, bQ, WV, bV,
                WdK, bdK, WdV, bdV):
    eblk = pl.BlockSpec((_EDGE_BLK, F), lambda i: (i, 0))
    dblk = pl.BlockSpec((_EDGE_BLK, 1), lambda i: (i, 0))
    wspec = pl.BlockSpec((F, F), lambda i: (0, 0))
    bspec = pl.BlockSpec((F,), lambda i: (0,))
    n_edges = xsrc.shape[0]
    return pl.pallas_call(
        _edge_body,
        grid=(n_edges // _EDGE_BLK,),
        in_specs=[dblk, eblk, eblk, bspec, wspec, bspec, wspec, bspec,
                  wspec, bspec, wspec, bspec, wspec, bspec],
        out_specs=[eblk, eblk],
        out_shape=[jax.ShapeDtypeStruct((n_edges, F), jnp.float32),
                   jax.ShapeDtypeStruct((n_edges, F), jnp.float32)],
    )(edge_d[:, None], xsrc, xdst, mu, WK, bK, WQ, bQ, WV, bV,
      WdK, bdK, WdV, bdV)


_NPAD = 10240
_NSUB = _NPAD // 16
_SITERS = -(-_NCHUNK // 16)


def _make_scatter_body(nchunk):
    siters = -(-nchunk // 16)

    def _scatter_body(logabs_hbm, neg_hbm, dst_hbm, zf_hbm,
                      acc_out,
                      idx, lbuf, acc):
        core = lax.axis_index("c")
        sub = lax.axis_index("s")
        pltpu.sync_copy(zf_hbm.at[pl.ds(sub * _NSUB, _NSUB)],
                        acc.at[pl.ds(sub * _NSUB, _NSUB)])
        plsc.subcore_barrier()

        def body(i, carry):
            c = i * 16 + sub

            @pl.when(c < nchunk)
            def _():
                base = c * _CH
                pltpu.sync_copy(dst_hbm.at[pl.ds(base, _CH)], idx)

                @pl.when(core == 0)
                def _():
                    pltpu.sync_copy(logabs_hbm.at[pl.ds(base, _CH)], lbuf)

                @pl.when(core == 1)
                def _():
                    pltpu.sync_copy(neg_hbm.at[pl.ds(base, _CH)], lbuf)

                pltpu.sync_copy(lbuf, acc.at[idx], add=True)

            return carry

        lax.fori_loop(0, siters, body, 0)
        plsc.subcore_barrier()
        pltpu.sync_copy(acc.at[pl.ds(sub * _NSUB, _NSUB)],
                        acc_out.at[core, pl.ds(sub * _NSUB, _NSUB)])

    return _scatter_body


def _sc_scatter(logabs, neg, dst, zf):
    n_edges = dst.shape[0]
    mesh = plsc.VectorSubcoreMesh(core_axis_name="c", subcore_axis_name="s")
    return pl.kernel(
        _make_scatter_body(n_edges // _CH),
        mesh=mesh,
        out_type=jax.ShapeDtypeStruct((2, _NPAD, F), jnp.float32),
        scratch_types=[
            pltpu.VMEM((_CH,), jnp.int32),
            pltpu.VMEM((_CH, F), jnp.float32),
            pltpu.VMEM_SHARED((_NPAD, F), jnp.float32),
        ],
    )(logabs, neg, dst, zf)


def _make_final_body(nparts):
    def _final_body(*refs):
        ls_refs = refs[:nparts]
        cnt_refs = refs[nparts:2 * nparts]
        wib, bib, y_ref = refs[2 * nparts:]
        ls = ls_refs[0][...]
        cnt = cnt_refs[0][...]
        for r in ls_refs[1:]:
            ls = ls + r[...]
        for r in cnt_refs[1:]:
            cnt = cnt + r[...]
        parity = cnt - 2.0 * jnp.floor(cnt * 0.5)
        sign = 1.0 - 2.0 * parity
        h = sign * jnp.exp(ls)
        y_ref[...] = h @ wib[...] + bib[...]

    return _final_body


def _final_stage(accs, WIB, bIB):
    nparts = len(accs)
    blk = pl.BlockSpec((_NODE_BLK, F), lambda i: (i, 0))
    wspec = pl.BlockSpec((F, F), lambda i: (0, 0))
    bspec = pl.BlockSpec((F,), lambda i: (0,))
    args = ([a[0, :N_NODES] for a in accs] + [a[1, :N_NODES] for a in accs]
            + [WIB, bIB])
    return pl.pallas_call(
        _make_final_body(nparts),
        grid=(N_NODES // _NODE_BLK,),
        in_specs=[blk] * (2 * nparts) + [wspec, bspec],
        out_specs=blk,
        out_shape=jax.ShapeDtypeStruct((N_NODES, F), jnp.float32),
    )(*args)


def kernel(x, edge_index, edge_d, WK, bK, WQ, bQ, WV, bV, WdK, bdK, WdV, bdV,
           WIB, bIB, WS1, bS1, WS2, bS2):
    src = edge_index[0]
    dst = edge_index[1]
    s1, s2 = _node_matmuls(x, WV, bV, WS1, bS1, WS2, bS2)

    mu = jnp.linspace(0.0, CUTOFF, F)
    zf = jnp.zeros((_NPAD, F), jnp.float32)
    part = N_EDGES // _NPARTS
    acc_parts = []
    for lo in range(0, N_EDGES, part):
        s_h = lax.dynamic_slice_in_dim(src, lo, part)
        d_h = lax.dynamic_slice_in_dim(dst, lo, part)
        e_h = lax.dynamic_slice_in_dim(edge_d, lo, part)
        xsrc, xdst = _sc_gather(x, s_h, d_h)
        logabs, neg = _edge_stage(e_h, xsrc, xdst, mu, WK, bK, WQ, bQ,
                                  WV, bV, WdK, bdK, WdV, bdV)
        acc_parts.append(_sc_scatter(logabs, neg, d_h, zf))
    y = _final_stage(acc_parts, WIB, bIB)
    return (y, s1, s2)
